# baseline (speedup 1.0000x reference)
import functools

import jax
import jax.numpy as jnp
from jax import lax
from jax.experimental import pallas as pl
from jax.experimental.pallas import tpu as pltpu
from jax.experimental.pallas import tpu_sc as plsc

T, N, F_IN, F_ST, H, E = 12, 10000, 16, 8, 128, 320000
HD2 = H // 2

NC, NS = 2, 16
NW = NC * NS
EPW = E // NW
SC_B = 40
SC_NB = EPW // SC_B
RPT8 = 1000


def _make_spmm(npass):
    mesh = plsc.VectorSubcoreMesh(
        core_axis_name="c", subcore_axis_name="s", num_cores=NC, num_subcores=NS)

    @functools.partial(
        pl.kernel,
        out_type=jax.ShapeDtypeStruct((NC, N, npass * H), jnp.float32),
        mesh=mesh,
        scratch_types=[
            pltpu.VMEM((EPW,), jnp.int32),
            pltpu.VMEM((EPW + 16,), jnp.float32),
            pltpu.VMEM((EPW,), jnp.int32),
            pltpu.VMEM((SC_B, H), jnp.float32),
            pltpu.VMEM((SC_B, H), jnp.float32),
            pltpu.VMEM((SC_B, H), jnp.float32),
            pltpu.VMEM_SHARED((N, H), jnp.float32),
            pltpu.SemaphoreType.DMA,
            pltpu.SemaphoreType.DMA,
            pltpu.SemaphoreType.DMA,
            pltpu.SemaphoreType.DMA,
            pltpu.SemaphoreType.DMA,
            pltpu.SemaphoreType.DMA,
            pltpu.SemaphoreType.DMA,
        ],
    )
    def spmm(*refs):
        xs = refs[:npass]
        rows_hbm, cols_hbm, vals_hbm, zeros_hbm, out_hbm = refs[npass:npass + 5]
        sc_refs = refs[npass + 5:]
        colv, valv, dstm = sc_refs[0:3]
        ring = sc_refs[3:6]
        acc = sc_refs[6]
        gsem = sc_refs[7:10]
        ssem = sc_refs[10:13]
        psem = sc_refs[13]
        c = lax.axis_index("c")
        s = lax.axis_index("s")
        w = c * NS + s
        z0 = pl.multiple_of(s * RPT8, 8)
        e0 = pl.multiple_of(w * EPW, 8)

        pre = [
            pltpu.make_async_copy(cols_hbm.at[pl.ds(e0, EPW)], colv, psem),
            pltpu.make_async_copy(vals_hbm.at[pl.ds(e0, EPW)],
                                  valv.at[pl.ds(0, EPW)], psem),
            pltpu.make_async_copy(rows_hbm.at[pl.ds(e0, EPW)], dstm, psem),
        ]
        for x in pre:
            x.start()
        for x in pre:
            x.wait()

        for p in range(npass):
            x_ref = xs[p]

            @pl.when(s < N // RPT8)
            def _zero():
                pltpu.sync_copy(zeros_hbm.at[pl.ds(0, RPT8)],
                                acc.at[pl.ds(z0, RPT8)])

            plsc.subcore_barrier()

            def gather(i, b):
                pltpu.async_copy(
                    x_ref.at[colv.at[pl.ds(i * SC_B, SC_B)]], ring[b], gsem[b])

            def gwait(b):
                pltpu.make_async_copy(
                    x_ref.at[colv.at[pl.ds(0, SC_B)]], ring[b], gsem[b]).wait()

            def swait(b):
                pltpu.make_async_copy(
                    ring[b], acc.at[dstm.at[pl.ds(0, SC_B)]], ssem[b]).wait()

            def step(i, b, first2=False, last=False):
                gwait(b)
                nb = (b + 1) % 3
                if not last:
                    if not first2:
                        swait(nb)
                    gather(i + 1, nb)

                def scale16(g16, carry2):
                    val16 = valv[pl.ds(i * SC_B + g16 * 16, 16)]
                    for k in range(16):
                        v = val16[k]
                        for q in range(H // 16):
                            ring[b][g16 * 16 + k, pl.ds(16 * q, 16)] = (
                                ring[b][g16 * 16 + k, pl.ds(16 * q, 16)] * v)
                    return carry2

                lax.fori_loop(0, SC_B // 16, scale16, 0)

                g0 = SC_B - SC_B % 16
                if SC_B % 16:
                    val16 = valv[pl.ds(i * SC_B + g0, 16)]
                    for k in range(SC_B % 16):
                        v = val16[k]
                        for q in range(H // 16):
                            ring[b][g0 + k, pl.ds(16 * q, 16)] = (
                                ring[b][g0 + k, pl.ds(16 * q, 16)] * v)

                pltpu.async_copy(ring[b],
                                 acc.at[dstm.at[pl.ds(i * SC_B, SC_B)]],
                                 ssem[b], add=True)

            gather(0, 0)
            step(0, 0, first2=True)
            step(1, 1, first2=True)

            def tri(k2, carry):
                i = 2 + 3 * k2
                step(i, 2)
                step(i + 1, 0)
                step(i + 2, 1)
                return carry

            lax.fori_loop(0, (SC_NB - 4) // 3, tri, 0)
            step(SC_NB - 2, 2)
            step(SC_NB - 1, 0, last=True)
            swait(0)
            swait(1)
            swait(2)
            plsc.subcore_barrier()

            @pl.when(s < N // RPT8)
            def _copy_out():
                pltpu.sync_copy(acc.at[pl.ds(z0, RPT8)],
                                out_hbm.at[c, pl.ds(z0, RPT8), pl.ds(p * H, H)])

            plsc.subcore_barrier()

    return spmm


_spmm1 = _make_spmm(1)
_spmm2 = _make_spmm(2)

RB = 1000
NRB = N // RB


def _ln_blk(x, g, b):
    mu = jnp.mean(x, axis=-1, keepdims=True)
    var = jnp.mean((x - mu) ** 2, axis=-1, keepdims=True)
    return g * (x - mu) * lax.rsqrt(var + 1e-5) + b


def _fuse_body(ns_ref, st_ref, wf_ref, bf_ref, gf_ref, bfn_ref, o_ref):
    ns = ns_ref[0]
    st = st_ref[...]
    y = (jnp.dot(ns, wf_ref[:F_IN, :], preferred_element_type=jnp.float32)
         + jnp.dot(st, wf_ref[F_IN:, :], preferred_element_type=jnp.float32)
         + bf_ref[...])
    y = _ln_blk(y, gf_ref[...], bfn_ref[...])
    o_ref[0] = 0.5 * y * (1.0 + lax.erf(y / jnp.sqrt(2.0).astype(jnp.float32)))


def _fuse(node_seq, static_feat, Wf, bf, gf, bfn):
    return pl.pallas_call(
        _fuse_body,
        grid=(T, NRB),
        in_specs=[
            pl.BlockSpec((1, RB, F_IN), lambda t, i: (t, i, 0)),
            pl.BlockSpec((RB, F_ST), lambda t, i: (i, 0)),
            pl.BlockSpec((F_IN + F_ST, HD2), lambda t, i: (0, 0)),
            pl.BlockSpec((1, HD2), lambda t, i: (0, 0)),
            pl.BlockSpec((1, HD2), lambda t, i: (0, 0)),
            pl.BlockSpec((1, HD2), lambda t, i: (0, 0)),
        ],
        out_specs=pl.BlockSpec((1, RB, HD2), lambda t, i: (t, i, 0)),
        out_shape=jax.ShapeDtypeStruct((T, N, HD2), jnp.float32),
    )(node_seq, static_feat, Wf, bf.reshape(1, -1), gf.reshape(1, -1),
      bfn.reshape(1, -1))


def _gate_body(dx, x_ref, h_ref, wg_ref, bg_ref, lo_ref, hi_ref):
    g = (jnp.dot(x_ref[...], wg_ref[:dx, :], preferred_element_type=jnp.float32)
         + jnp.dot(h_ref[...], wg_ref[dx:, :], preferred_element_type=jnp.float32)
         + bg_ref[...])
    lo_ref[...] = g[:, :H]
    hi_ref[...] = g[:, H:]


def _gate(x, h, Wg, bg, dx):
    return pl.pallas_call(
        functools.partial(_gate_body, dx),
        grid=(NRB,),
        in_specs=[
            pl.BlockSpec((RB, dx), lambda i: (i, 0)),
            pl.BlockSpec((RB, H), lambda i: (i, 0)),
            pl.BlockSpec((dx + H, 2 * H), lambda i: (0, 0)),
            pl.BlockSpec((1, 2 * H), lambda i: (0, 0)),
        ],
        out_specs=[
            pl.BlockSpec((RB, H), lambda i: (i, 0)),
            pl.BlockSpec((RB, H), lambda i: (i, 0)),
        ],
        out_shape=[
            jax.ShapeDtypeStruct((N, H), jnp.float32),
            jax.ShapeDtypeStruct((N, H), jnp.float32),
        ],
    )(x, h, Wg, bg.reshape(1, -1))


def _mid_body(dx, gp_ref, x_ref, h_ref, wc_ref, bc_ref, c_ref, u_ref):
    r = jax.nn.sigmoid(gp_ref[0, :, :H] + gp_ref[1, :, :H])
    u = jax.nn.sigmoid(gp_ref[0, :, H:] + gp_ref[1, :, H:])
    rh = r * h_ref[...]
    c_ref[...] = (jnp.dot(x_ref[...], wc_ref[:dx, :], preferred_element_type=jnp.float32)
                  + jnp.dot(rh, wc_ref[dx:, :], preferred_element_type=jnp.float32)
                  + bc_ref[...])
    u_ref[...] = u


def _mid(gp, x, h, Wc, bc, dx):
    return pl.pallas_call(
        functools.partial(_mid_body, dx),
        grid=(NRB,),
        in_specs=[
            pl.BlockSpec((NC, RB, 2 * H), lambda i: (0, i, 0)),
            pl.BlockSpec((RB, dx), lambda i: (i, 0)),
            pl.BlockSpec((RB, H), lambda i: (i, 0)),
            pl.BlockSpec((dx + H, H), lambda i: (0, 0)),
            pl.BlockSpec((1, H), lambda i: (0, 0)),
        ],
        out_specs=[
            pl.BlockSpec((RB, H), lambda i: (i, 0)),
            pl.BlockSpec((RB, H), lambda i: (i, 0)),
        ],
        out_shape=[
            jax.ShapeDtypeStruct((N, H), jnp.float32),
            jax.ShapeDtypeStruct((N, H), jnp.float32),
        ],
    )(gp, x, h, Wc, bc.reshape(1, -1))


def _hn_from(cp_ref, u_ref, h_ref, gn_ref, bn_ref):
    c = jnp.tanh(cp_ref[0] + cp_ref[1])
    u = u_ref[...]
    z = u * h_ref[...] + (1.0 - u) * c
    return _ln_blk(z, gn_ref[...], bn_ref[...])


def _fin_body(cp_ref, u_ref, h_ref, gn_ref, bn_ref, hn_ref):
    hn_ref[...] = _hn_from(cp_ref, u_ref, h_ref, gn_ref, bn_ref)


def _fin(cp, u, h, gn, bn):
    return pl.pallas_call(
        _fin_body,
        grid=(NRB,),
        in_specs=[
            pl.BlockSpec((NC, RB, H), lambda i: (0, i, 0)),
            pl.BlockSpec((RB, H), lambda i: (i, 0)),
            pl.BlockSpec((RB, H), lambda i: (i, 0)),
            pl.BlockSpec((1, H), lambda i: (0, 0)),
            pl.BlockSpec((1, H), lambda i: (0, 0)),
        ],
        out_specs=pl.BlockSpec((RB, H), lambda i: (i, 0)),
        out_shape=jax.ShapeDtypeStruct((N, H), jnp.float32),
    )(cp, u, h, gn.reshape(1, -1), bn.reshape(1, -1))


def _fingate_body(dx, use_hn_as_x, cp_ref, u_ref, h_ref, gn_ref, bn_ref,
                  xn_ref, hx_ref, wg_ref, bg_ref, hn_ref, lo_ref, hi_ref):
    hn = _hn_from(cp_ref, u_ref, h_ref, gn_ref, bn_ref)
    hn_ref[...] = hn
    x = hn if use_hn_as_x else xn_ref[...]
    g = (jnp.dot(x, wg_ref[:dx, :], preferred_element_type=jnp.float32)
         + jnp.dot(hx_ref[...], wg_ref[dx:, :], preferred_element_type=jnp.float32)
         + bg_ref[...])
    lo_ref[...] = g[:, :H]
    hi_ref[...] = g[:, H:]


def _fingate(cp, u, h, gn, bn, xn, hx, Wg, bg, dx, use_hn_as_x):
    return pl.pallas_call(
        functools.partial(_fingate_body, dx, use_hn_as_x),
        grid=(NRB,),
        in_specs=[
            pl.BlockSpec((NC, RB, H), lambda i: (0, i, 0)),
            pl.BlockSpec((RB, H), lambda i: (i, 0)),
            pl.BlockSpec((RB, H), lambda i: (i, 0)),
            pl.BlockSpec((1, H), lambda i: (0, 0)),
            pl.BlockSpec((1, H), lambda i: (0, 0)),
            pl.BlockSpec((RB, dx), lambda i: (i, 0)),
            pl.BlockSpec((RB, H), lambda i: (i, 0)),
            pl.BlockSpec((dx + H, 2 * H), lambda i: (0, 0)),
            pl.BlockSpec((1, 2 * H), lambda i: (0, 0)),
        ],
        out_specs=[
            pl.BlockSpec((RB, H), lambda i: (i, 0)),
            pl.BlockSpec((RB, H), lambda i: (i, 0)),
            pl.BlockSpec((RB, H), lambda i: (i, 0)),
        ],
        out_shape=[
            jax.ShapeDtypeStruct((N, H), jnp.float32),
            jax.ShapeDtypeStruct((N, H), jnp.float32),
            jax.ShapeDtypeStruct((N, H), jnp.float32),
        ],
    )(cp, u, h, gn.reshape(1, -1), bn.reshape(1, -1), xn, hx, Wg,
      bg.reshape(1, -1))


def kernel(node_seq, static_feat, adj_vals, Wf, bf, gf, bfn, Wg0, bg0, Wc0, bc0,
           gn0, bn0, Wg1, bg1, Wc1, bc1, gn1, bn1, edge_index):
    rows3d = edge_index[0]
    cols = edge_index[1]
    zeros = jnp.zeros((RPT8, H), jnp.float32)

    xall = _fuse(node_seq, static_feat, Wf, bf, gf, bfn)
    h0 = jnp.zeros((N, H), jnp.float32)
    h1 = jnp.zeros((N, H), jnp.float32)
    glo, ghi = _gate(xall[0], h0, Wg0, bg0, HD2)
    for t in range(T):
        gp = _spmm2(glo, ghi, rows3d, cols, adj_vals, zeros)
        cpre, u = _mid(gp, xall[t], h0, Wc0, bc0, HD2)
        cp = _spmm1(cpre, rows3d, cols, adj_vals, zeros)
        h0, glo, ghi = _fingate(cp, u, h0, gn0, bn0, h0, h1, Wg1, bg1, H, True)
        gp = _spmm2(glo, ghi, rows3d, cols, adj_vals, zeros)
        cpre, u = _mid(gp, h0, h1, Wc1, bc1, H)
        cp = _spmm1(cpre, rows3d, cols, adj_vals, zeros)
        if t < T - 1:
            h1, glo, ghi = _fingate(cp, u, h1, gn1, bn1, xall[t + 1], h0,
                                    Wg0, bg0, HD2, False)
        else:
            h1 = _fin(cp, u, h1, gn1, bn1)
    return jnp.stack([h0, h1])


# speedup vs baseline: 1.0038x; 1.0038x over previous
import functools

import jax
import jax.numpy as jnp
from jax import lax
from jax.experimental import pallas as pl
from jax.experimental.pallas import tpu as pltpu
from jax.experimental.pallas import tpu_sc as plsc

T, N, F_IN, F_ST, H, E = 12, 10000, 16, 8, 128, 320000
HD2 = H // 2

NC, NS = 2, 16
NW = NC * NS
EPW = E // NW
SC_B = 40
SC_NB = EPW // SC_B
RPT8 = 1000


def _make_spmm(npass):
    mesh = plsc.VectorSubcoreMesh(
        core_axis_name="c", subcore_axis_name="s", num_cores=NC, num_subcores=NS)

    @functools.partial(
        pl.kernel,
        out_type=jax.ShapeDtypeStruct((NC, N, npass * H), jnp.float32),
        mesh=mesh,
        scratch_types=[
            pltpu.VMEM((EPW,), jnp.int32),
            pltpu.VMEM((EPW + 16,), jnp.float32),
            pltpu.VMEM((EPW,), jnp.int32),
            pltpu.VMEM((SC_B, H), jnp.float32),
            pltpu.VMEM((SC_B, H), jnp.float32),
            pltpu.VMEM((SC_B, H), jnp.float32),
            pltpu.VMEM_SHARED((N, H), jnp.float32),
            pltpu.SemaphoreType.DMA,
            pltpu.SemaphoreType.DMA,
            pltpu.SemaphoreType.DMA,
            pltpu.SemaphoreType.DMA,
            pltpu.SemaphoreType.DMA,
            pltpu.SemaphoreType.DMA,
            pltpu.SemaphoreType.DMA,
        ],
    )
    def spmm(*refs):
        xs = refs[:npass]
        rows_hbm, cols_hbm, vals_hbm, zeros_hbm, out_hbm = refs[npass:npass + 5]
        sc_refs = refs[npass + 5:]
        colv, valv, dstm = sc_refs[0:3]
        ring = sc_refs[3:6]
        acc = sc_refs[6]
        gsem = sc_refs[7:10]
        ssem = sc_refs[10:13]
        psem = sc_refs[13]
        c = lax.axis_index("c")
        s = lax.axis_index("s")
        w = c * NS + s
        z0 = pl.multiple_of(s * RPT8, 8)
        e0 = pl.multiple_of(w * EPW, 8)

        pre = [
            pltpu.make_async_copy(cols_hbm.at[pl.ds(e0, EPW)], colv, psem),
            pltpu.make_async_copy(vals_hbm.at[pl.ds(e0, EPW)],
                                  valv.at[pl.ds(0, EPW)], psem),
            pltpu.make_async_copy(rows_hbm.at[pl.ds(e0, EPW)], dstm, psem),
        ]
        for x in pre:
            x.start()

        for p in range(npass):
            x_ref = xs[p]

            def gather(i, b):
                pltpu.async_copy(
                    x_ref.at[colv.at[pl.ds(i * SC_B, SC_B)]], ring[b], gsem[b])

            def gwait(b):
                pltpu.make_async_copy(
                    x_ref.at[colv.at[pl.ds(0, SC_B)]], ring[b], gsem[b]).wait()

            def swait(b):
                pltpu.make_async_copy(
                    ring[b], acc.at[dstm.at[pl.ds(0, SC_B)]], ssem[b]).wait()

            def step(i, b, first2=False, last=False):
                gwait(b)
                nb = (b + 1) % 3
                if not last:
                    if not first2:
                        swait(nb)
                    gather(i + 1, nb)

                def scale16(g16, carry2):
                    val16 = valv[pl.ds(i * SC_B + g16 * 16, 16)]
                    for k in range(16):
                        v = val16[k]
                        for q in range(H // 16):
                            ring[b][g16 * 16 + k, pl.ds(16 * q, 16)] = (
                                ring[b][g16 * 16 + k, pl.ds(16 * q, 16)] * v)
                    return carry2

                lax.fori_loop(0, SC_B // 16, scale16, 0)

                g0 = SC_B - SC_B % 16
                if SC_B % 16:
                    val16 = valv[pl.ds(i * SC_B + g0, 16)]
                    for k in range(SC_B % 16):
                        v = val16[k]
                        for q in range(H // 16):
                            ring[b][g0 + k, pl.ds(16 * q, 16)] = (
                                ring[b][g0 + k, pl.ds(16 * q, 16)] * v)

                pltpu.async_copy(ring[b],
                                 acc.at[dstm.at[pl.ds(i * SC_B, SC_B)]],
                                 ssem[b], add=True)

            if p == 0:
                for x in pre:
                    x.wait()
            gather(0, 0)

            @pl.when(s < N // RPT8)
            def _zero():
                pltpu.sync_copy(zeros_hbm.at[pl.ds(0, RPT8)],
                                acc.at[pl.ds(z0, RPT8)])

            plsc.subcore_barrier()

            step(0, 0, first2=True)
            step(1, 1, first2=True)

            def tri(k2, carry):
                i = 2 + 3 * k2
                step(i, 2)
                step(i + 1, 0)
                step(i + 2, 1)
                return carry

            lax.fori_loop(0, (SC_NB - 4) // 3, tri, 0)
            step(SC_NB - 2, 2)
            step(SC_NB - 1, 0, last=True)
            swait(0)
            swait(1)
            swait(2)
            plsc.subcore_barrier()

            @pl.when(s < N // RPT8)
            def _copy_out():
                pltpu.sync_copy(acc.at[pl.ds(z0, RPT8)],
                                out_hbm.at[c, pl.ds(z0, RPT8), pl.ds(p * H, H)])

            plsc.subcore_barrier()

    return spmm


_spmm1 = _make_spmm(1)
_spmm2 = _make_spmm(2)

RB = 1000
NRB = N // RB


def _ln_blk(x, g, b):
    mu = jnp.mean(x, axis=-1, keepdims=True)
    var = jnp.mean((x - mu) ** 2, axis=-1, keepdims=True)
    return g * (x - mu) * lax.rsqrt(var + 1e-5) + b


def _fuse_body(ns_ref, st_ref, wf_ref, bf_ref, gf_ref, bfn_ref, o_ref):
    ns = ns_ref[0]
    st = st_ref[...]
    y = (jnp.dot(ns, wf_ref[:F_IN, :], preferred_element_type=jnp.float32)
         + jnp.dot(st, wf_ref[F_IN:, :], preferred_element_type=jnp.float32)
         + bf_ref[...])
    y = _ln_blk(y, gf_ref[...], bfn_ref[...])
    o_ref[0] = 0.5 * y * (1.0 + lax.erf(y / jnp.sqrt(2.0).astype(jnp.float32)))


def _fuse(node_seq, static_feat, Wf, bf, gf, bfn):
    return pl.pallas_call(
        _fuse_body,
        grid=(T, NRB),
        in_specs=[
            pl.BlockSpec((1, RB, F_IN), lambda t, i: (t, i, 0)),
            pl.BlockSpec((RB, F_ST), lambda t, i: (i, 0)),
            pl.BlockSpec((F_IN + F_ST, HD2), lambda t, i: (0, 0)),
            pl.BlockSpec((1, HD2), lambda t, i: (0, 0)),
            pl.BlockSpec((1, HD2), lambda t, i: (0, 0)),
            pl.BlockSpec((1, HD2), lambda t, i: (0, 0)),
        ],
        out_specs=pl.BlockSpec((1, RB, HD2), lambda t, i: (t, i, 0)),
        out_shape=jax.ShapeDtypeStruct((T, N, HD2), jnp.float32),
    )(node_seq, static_feat, Wf, bf.reshape(1, -1), gf.reshape(1, -1),
      bfn.reshape(1, -1))


def _gate_body(dx, x_ref, h_ref, wg_ref, bg_ref, lo_ref, hi_ref):
    g = (jnp.dot(x_ref[...], wg_ref[:dx, :], preferred_element_type=jnp.float32)
         + jnp.dot(h_ref[...], wg_ref[dx:, :], preferred_element_type=jnp.float32)
         + bg_ref[...])
    lo_ref[...] = g[:, :H]
    hi_ref[...] = g[:, H:]


def _gate(x, h, Wg, bg, dx):
    return pl.pallas_call(
        functools.partial(_gate_body, dx),
        grid=(NRB,),
        in_specs=[
            pl.BlockSpec((RB, dx), lambda i: (i, 0)),
            pl.BlockSpec((RB, H), lambda i: (i, 0)),
            pl.BlockSpec((dx + H, 2 * H), lambda i: (0, 0)),
            pl.BlockSpec((1, 2 * H), lambda i: (0, 0)),
        ],
        out_specs=[
            pl.BlockSpec((RB, H), lambda i: (i, 0)),
            pl.BlockSpec((RB, H), lambda i: (i, 0)),
        ],
        out_shape=[
            jax.ShapeDtypeStruct((N, H), jnp.float32),
            jax.ShapeDtypeStruct((N, H), jnp.float32),
        ],
    )(x, h, Wg, bg.reshape(1, -1))


def _mid_body(dx, gp_ref, x_ref, h_ref, wc_ref, bc_ref, c_ref, u_ref):
    r = jax.nn.sigmoid(gp_ref[0, :, :H] + gp_ref[1, :, :H])
    u = jax.nn.sigmoid(gp_ref[0, :, H:] + gp_ref[1, :, H:])
    rh = r * h_ref[...]
    c_ref[...] = (jnp.dot(x_ref[...], wc_ref[:dx, :], preferred_element_type=jnp.float32)
                  + jnp.dot(rh, wc_ref[dx:, :], preferred_element_type=jnp.float32)
                  + bc_ref[...])
    u_ref[...] = u


def _mid(gp, x, h, Wc, bc, dx):
    return pl.pallas_call(
        functools.partial(_mid_body, dx),
        grid=(NRB,),
        in_specs=[
            pl.BlockSpec((NC, RB, 2 * H), lambda i: (0, i, 0)),
            pl.BlockSpec((RB, dx), lambda i: (i, 0)),
            pl.BlockSpec((RB, H), lambda i: (i, 0)),
            pl.BlockSpec((dx + H, H), lambda i: (0, 0)),
            pl.BlockSpec((1, H), lambda i: (0, 0)),
        ],
        out_specs=[
            pl.BlockSpec((RB, H), lambda i: (i, 0)),
            pl.BlockSpec((RB, H), lambda i: (i, 0)),
        ],
        out_shape=[
            jax.ShapeDtypeStruct((N, H), jnp.float32),
            jax.ShapeDtypeStruct((N, H), jnp.float32),
        ],
    )(gp, x, h, Wc, bc.reshape(1, -1))


def _hn_from(cp_ref, u_ref, h_ref, gn_ref, bn_ref):
    c = jnp.tanh(cp_ref[0] + cp_ref[1])
    u = u_ref[...]
    z = u * h_ref[...] + (1.0 - u) * c
    return _ln_blk(z, gn_ref[...], bn_ref[...])


def _fin_body(cp_ref, u_ref, h_ref, gn_ref, bn_ref, hn_ref):
    hn_ref[...] = _hn_from(cp_ref, u_ref, h_ref, gn_ref, bn_ref)


def _fin(cp, u, h, gn, bn):
    return pl.pallas_call(
        _fin_body,
        grid=(NRB,),
        in_specs=[
            pl.BlockSpec((NC, RB, H), lambda i: (0, i, 0)),
            pl.BlockSpec((RB, H), lambda i: (i, 0)),
            pl.BlockSpec((RB, H), lambda i: (i, 0)),
            pl.BlockSpec((1, H), lambda i: (0, 0)),
            pl.BlockSpec((1, H), lambda i: (0, 0)),
        ],
        out_specs=pl.BlockSpec((RB, H), lambda i: (i, 0)),
        out_shape=jax.ShapeDtypeStruct((N, H), jnp.float32),
    )(cp, u, h, gn.reshape(1, -1), bn.reshape(1, -1))


def _fingate_body(dx, use_hn_as_x, cp_ref, u_ref, h_ref, gn_ref, bn_ref,
                  xn_ref, hx_ref, wg_ref, bg_ref, hn_ref, lo_ref, hi_ref):
    hn = _hn_from(cp_ref, u_ref, h_ref, gn_ref, bn_ref)
    hn_ref[...] = hn
    x = hn if use_hn_as_x else xn_ref[...]
    g = (jnp.dot(x, wg_ref[:dx, :], preferred_element_type=jnp.float32)
         + jnp.dot(hx_ref[...], wg_ref[dx:, :], preferred_element_type=jnp.float32)
         + bg_ref[...])
    lo_ref[...] = g[:, :H]
    hi_ref[...] = g[:, H:]


def _fingate(cp, u, h, gn, bn, xn, hx, Wg, bg, dx, use_hn_as_x):
    return pl.pallas_call(
        functools.partial(_fingate_body, dx, use_hn_as_x),
        grid=(NRB,),
        in_specs=[
            pl.BlockSpec((NC, RB, H), lambda i: (0, i, 0)),
            pl.BlockSpec((RB, H), lambda i: (i, 0)),
            pl.BlockSpec((RB, H), lambda i: (i, 0)),
            pl.BlockSpec((1, H), lambda i: (0, 0)),
            pl.BlockSpec((1, H), lambda i: (0, 0)),
            pl.BlockSpec((RB, dx), lambda i: (i, 0)),
            pl.BlockSpec((RB, H), lambda i: (i, 0)),
            pl.BlockSpec((dx + H, 2 * H), lambda i: (0, 0)),
            pl.BlockSpec((1, 2 * H), lambda i: (0, 0)),
        ],
        out_specs=[
            pl.BlockSpec((RB, H), lambda i: (i, 0)),
            pl.BlockSpec((RB, H), lambda i: (i, 0)),
            pl.BlockSpec((RB, H), lambda i: (i, 0)),
        ],
        out_shape=[
            jax.ShapeDtypeStruct((N, H), jnp.float32),
            jax.ShapeDtypeStruct((N, H), jnp.float32),
            jax.ShapeDtypeStruct((N, H), jnp.float32),
        ],
    )(cp, u, h, gn.reshape(1, -1), bn.reshape(1, -1), xn, hx, Wg,
      bg.reshape(1, -1))


def kernel(node_seq, static_feat, adj_vals, Wf, bf, gf, bfn, Wg0, bg0, Wc0, bc0,
           gn0, bn0, Wg1, bg1, Wc1, bc1, gn1, bn1, edge_index):
    rows3d = edge_index[0]
    cols = edge_index[1]
    zeros = jnp.zeros((RPT8, H), jnp.float32)

    xall = _fuse(node_seq, static_feat, Wf, bf, gf, bfn)
    h0 = jnp.zeros((N, H), jnp.float32)
    h1 = jnp.zeros((N, H), jnp.float32)
    glo, ghi = _gate(xall[0], h0, Wg0, bg0, HD2)
    for t in range(T):
        gp = _spmm2(glo, ghi, rows3d, cols, adj_vals, zeros)
        cpre, u = _mid(gp, xall[t], h0, Wc0, bc0, HD2)
        cp = _spmm1(cpre, rows3d, cols, adj_vals, zeros)
        h0, glo, ghi = _fingate(cp, u, h0, gn0, bn0, h0, h1, Wg1, bg1, H, True)
        gp = _spmm2(glo, ghi, rows3d, cols, adj_vals, zeros)
        cpre, u = _mid(gp, h0, h1, Wc1, bc1, H)
        cp = _spmm1(cpre, rows3d, cols, adj_vals, zeros)
        if t < T - 1:
            h1, glo, ghi = _fingate(cp, u, h1, gn1, bn1, xall[t + 1], h0,
                                    Wg0, bg0, HD2, False)
        else:
            h1 = _fin(cp, u, h1, gn1, bn1)
    return jnp.stack([h0, h1])
